# baseline (speedup 1.0000x reference)
import functools

import jax
import jax.numpy as jnp
from jax import lax
from jax.experimental import pallas as pl
from jax.experimental.pallas import tpu as pltpu
from jax.experimental.pallas import tpu_sc as plsc

NC = 2
NS = 16
LANES = 16
CHUNK = 64
NBUF = 4
IGR = 3
GPC = 16
GPT0 = 14
GPT1 = 6
CH0 = GPC * GPT0
CH1 = GPC * GPT1
PADR = 128


@functools.partial(jax.jit, static_argnames=("N", "D"))
def _sc_edge_aggregate(x, src_g, dst_g, zrows, *, N, D):
    NP = N + PADR
    STRIDE = 624
    SPAN = 640
    assert STRIDE * (NS - 1) + SPAN == N

    mesh = plsc.VectorSubcoreMesh(
        core_axis_name="c", subcore_axis_name="s", num_cores=NC,
        num_subcores=NS)

    @functools.partial(
        pl.kernel,
        out_type=[
            jax.ShapeDtypeStruct((NC, N, D), jnp.float32),
            jax.ShapeDtypeStruct((NC * NS, 1, N), jnp.float32),
        ],
        mesh=mesh,
        compiler_params=pltpu.CompilerParams(needs_layout_passes=False),
        scratch_types=[
            pltpu.VMEM((IGR, 8, 2 * CHUNK), jnp.int32),
            pltpu.VMEM((IGR, 8, 2 * CHUNK), jnp.int32),
            pltpu.VMEM((NBUF, CHUNK, D), jnp.float32),
            pltpu.VMEM((NBUF, CHUNK), jnp.int32),
            pltpu.VMEM((NP,), jnp.float32),
            pltpu.VMEM_SHARED((NP, D), jnp.float32),
            pltpu.SemaphoreType.DMA((IGR,)),
            pltpu.SemaphoreType.DMA((IGR,)),
            pltpu.SemaphoreType.DMA((NBUF,)),
            pltpu.SemaphoreType.DMA((NBUF,)),
        ],
    )
    def k(x_hbm, src_hbm, dst_hbm, z_hbm, agg_out, deg_out,
          sring, dring, rows_v, dstb_v, deg_v, agg_sh,
          gssem, gdsem, gsem, ssem):
        c = lax.axis_index("c")
        s = lax.axis_index("s")
        wid = c * NS + s
        CHL = jnp.where(c == 0, CH0, CH1)
        GRPL = jnp.where(c == 0, GPT0, GPT1)
        gbase = jnp.where(c == 0, s * GPT0, NS * GPT0 + s * GPT1)

        pltpu.sync_copy(z_hbm, agg_sh.at[pl.ds(s * STRIDE, SPAN)])
        zeros16 = jnp.zeros((LANES,), jnp.float32)

        @pl.loop(0, NP // LANES)
        def _(i):
            deg_v[pl.ds(i * LANES, LANES)] = zeros16

        plsc.subcore_barrier()

        ones16 = jnp.ones((LANES,), jnp.float32)

        def sgrp_desc(g):
            b = lax.rem(g, IGR)
            return pltpu.make_async_copy(
                src_hbm.at[gbase + g], sring.at[b], gssem.at[b])

        def dgrp_desc(g):
            b = lax.rem(g, IGR)
            return pltpu.make_async_copy(
                dst_hbm.at[gbase + g], dring.at[b], gdsem.at[b])

        def chunk_pos(j):
            gb = lax.rem(lax.div(j, GPC), IGR)
            r = lax.rem(lax.div(j, 2), 8)
            h = lax.rem(j, 2) * CHUNK
            return gb, r, h

        def gather_desc(j):
            b = lax.rem(j, NBUF)
            gb, r, h = chunk_pos(j)
            return pltpu.make_async_copy(
                x_hbm.at[sring.at[gb, r, pl.ds(h, CHUNK)]], rows_v.at[b],
                gsem.at[b])

        def scatter_desc(j):
            b = lax.rem(j, NBUF)
            return pltpu.make_async_copy(
                rows_v.at[b], agg_sh.at[dstb_v.at[b]], ssem.at[b])

        sgrp_desc(0).start()
        dgrp_desc(0).start()
        sgrp_desc(1).start()
        dgrp_desc(1).start()
        sgrp_desc(0).wait()
        dgrp_desc(0).wait()
        for j in range(NBUF - 1):
            gather_desc(j).start()

        @pl.loop(0, CHL)
        def _(j):
            nxt = j + NBUF - 1

            @pl.when(nxt < CHL)
            def _():
                @pl.when(j >= 1)
                def _():
                    scatter_desc(j - 1).wait()

                gn = lax.div(nxt, GPC)

                @pl.when(lax.rem(nxt, GPC) == 0)
                def _():
                    sgrp_desc(gn).wait()
                    dgrp_desc(gn).wait()

                    @pl.when(gn + 1 < GRPL)
                    def _():
                        sgrp_desc(gn + 1).start()
                        dgrp_desc(gn + 1).start()

                gather_desc(nxt).start()

            gather_desc(j).wait()
            b = lax.rem(j, NBUF)
            gb, r, h = chunk_pos(j)
            for kk in range(CHUNK // LANES):
                dvec = dring[gb, r, pl.ds(h + kk * LANES, LANES)]
                dstb_v[b, pl.ds(kk * LANES, LANES)] = dvec
                plsc.addupdate_scatter(deg_v, [dvec], ones16)
            scatter_desc(j).start(add=True)

        for i in range(NBUF):
            scatter_desc(CHL - NBUF + i).wait()

        plsc.subcore_barrier()

        pltpu.sync_copy(agg_sh.at[pl.ds(s * STRIDE, SPAN)],
                        agg_out.at[c, pl.ds(s * STRIDE, SPAN)])
        pltpu.sync_copy(deg_v.at[pl.ds(0, N)], deg_out.at[wid, 0])

    return k(x, src_g, dst_g, zrows)


def _tc_body(R, G, grid,
             x_r, a0_r, a1_r, deg_r, batch_r, wl_r, wr_r, bl_r, gm_r, bt_r,
             node_r, graph_r, acc_r):
    i = pl.program_id(0)
    deg = jnp.sum(deg_r[...], axis=1, keepdims=True)
    agg = a0_r[...] + a1_r[...]
    mean = agg / jnp.maximum(deg, 1.0)
    conv = (lax.dot_general(mean, wl_r[...], (((1,), (1,)), ((), ())))
            + bl_r[...]
            + lax.dot_general(x_r[...], wr_r[...], (((1,), (1,)), ((), ()))))

    def ln(v):
        mu = jnp.mean(v, axis=-1, keepdims=True)
        var = jnp.mean((v - mu) * (v - mu), axis=-1, keepdims=True)
        return (v - mu) / jnp.sqrt(var + 1e-5) * gm_r[...] + bt_r[...]

    node_r[...] = jnp.maximum(ln(conv + x_r[...]), 0.0)

    oh = (batch_r[...] == lax.broadcasted_iota(jnp.int32, (R, G), 1)
          ).astype(jnp.float32)
    contrib = lax.dot_general(oh, conv, (((0,), (0,)), ((), ())))

    @pl.when(i == 0)
    def _():
        acc_r[...] = contrib

    @pl.when(i > 0)
    def _():
        acc_r[...] = acc_r[...] + contrib

    @pl.when(i == grid - 1)
    def _():
        graph_r[...] = jnp.maximum(ln(acc_r[...]), 0.0)


@functools.partial(jax.jit, static_argnames=("R", "G"))
def _tc_tail(x, a0, a1, deg_p, batch2, W_l, W_r, b_l, gamma, beta, *, R, G):
    N, D = x.shape
    grid = N // R
    row_spec = pl.BlockSpec((R, D), lambda i: (i, 0))
    full_spec = pl.BlockSpec((D, D), lambda i: (0, 0))
    vec_spec = pl.BlockSpec((1, D), lambda i: (0, 0))
    return pl.pallas_call(
        functools.partial(_tc_body, R, G, grid),
        grid=(grid,),
        in_specs=[
            row_spec,
            row_spec,
            row_spec,
            pl.BlockSpec((R, NC * NS), lambda i: (i, 0)),
            pl.BlockSpec((R, 1), lambda i: (i, 0)),
            full_spec,
            full_spec,
            vec_spec,
            vec_spec,
            vec_spec,
        ],
        out_specs=[
            row_spec,
            pl.BlockSpec((G, D), lambda i: (0, 0)),
        ],
        out_shape=[
            jax.ShapeDtypeStruct((N, D), jnp.float32),
            jax.ShapeDtypeStruct((G, D), jnp.float32),
        ],
        scratch_shapes=[pltpu.VMEM((G, D), jnp.float32)],
    )(x, a0, a1, deg_p, batch2, W_l, W_r, b_l, gamma, beta)


def kernel(x, edge_index, batch, W_l, b_l, W_r, gamma, beta):
    N, D = x.shape
    E = edge_index.shape[1]
    G = 16

    src = edge_index[0].astype(jnp.int32)
    dst = edge_index[1].astype(jnp.int32)

    ngroups = NS * (GPT0 + GPT1)
    E_pad = ngroups * 8 * 2 * CHUNK
    pad = E_pad - E
    assert pad >= 0
    src_p = jnp.concatenate([src, jnp.zeros((pad,), jnp.int32)])
    dst_p = jnp.concatenate(
        [dst, N + jnp.arange(pad, dtype=jnp.int32) % PADR])
    src_g = src_p.reshape(ngroups, 8, 2 * CHUNK)
    dst_g = dst_p.reshape(ngroups, 8, 2 * CHUNK)
    zrows = jnp.zeros((640, D), jnp.float32)

    agg_part, deg_part = _sc_edge_aggregate(x, src_g, dst_g, zrows, N=N, D=D)

    node_out, graph_out = _tc_tail(
        x, agg_part[0], agg_part[1], deg_part.reshape(NC * NS, N).T,
        batch.astype(jnp.int32).reshape(N, 1),
        W_l, W_r, b_l.reshape(1, D), gamma.reshape(1, D), beta.reshape(1, D),
        R=1000, G=G)
    return (node_out, graph_out)


# speedup vs baseline: 3.0797x; 3.0797x over previous
import functools

import jax
import jax.numpy as jnp
from jax import lax
from jax.experimental import pallas as pl
from jax.experimental.pallas import tpu as pltpu
from jax.experimental.pallas import tpu_sc as plsc

NC = 2
NS = 16
LANES = 16
CHUNK = 64
NBUF = 4
IRING = 8
FPD = 4
CH0 = 200
CH1 = 112


@functools.partial(jax.jit, static_argnames=("N", "D"))
def _sc_edge_aggregate(x, edge_index, zrows, *, N, D):
    NP = N
    STRIDE = 624
    SPAN = 640
    assert STRIDE * (NS - 1) + SPAN == N

    mesh = plsc.VectorSubcoreMesh(
        core_axis_name="c", subcore_axis_name="s", num_cores=NC,
        num_subcores=NS)

    @functools.partial(
        pl.kernel,
        out_type=[
            jax.ShapeDtypeStruct((NC, N, D), jnp.float32),
            jax.ShapeDtypeStruct((NC * NS, 1, N), jnp.float32),
        ],
        mesh=mesh,
        compiler_params=pltpu.CompilerParams(needs_layout_passes=False),
        scratch_types=[
            pltpu.VMEM((IRING, 2, 2 * CHUNK), jnp.int32),
            pltpu.VMEM((NBUF, CHUNK, D), jnp.float32),
            pltpu.VMEM((NBUF, CHUNK), jnp.int32),
            pltpu.VMEM((NP,), jnp.float32),
            pltpu.VMEM_SHARED((NP, D), jnp.float32),
            pltpu.SemaphoreType.DMA((IRING,)),
            pltpu.SemaphoreType.DMA((NBUF,)),
            pltpu.SemaphoreType.DMA((NBUF,)),
        ],
    )
    def k(x_hbm, edges_hbm, z_hbm, agg_out, deg_out,
          idx_v, rows_v, dstb_v, deg_v, agg_sh, isem, gsem, ssem):
        c = lax.axis_index("c")
        s = lax.axis_index("s")
        wid = c * NS + s
        CHL = jnp.where(c == 0, CH0, CH1 + 2 * (s < 4))
        fbase = jnp.where(c == 0, s * (CH0 // 2),
                          NS * CH0 // 2 + s * (CH1 // 2)
                          + jnp.minimum(s, 4))

        pltpu.sync_copy(z_hbm, agg_sh.at[pl.ds(s * STRIDE, SPAN)])
        zeros16 = jnp.zeros((LANES,), jnp.float32)

        @pl.loop(0, NP // LANES)
        def _(i):
            deg_v[pl.ds(i * LANES, LANES)] = zeros16

        plsc.subcore_barrier()

        ones16 = jnp.ones((LANES,), jnp.float32)

        def idx_desc(f):
            b = lax.rem(f, IRING)
            return pltpu.make_async_copy(
                edges_hbm.at[:, pl.ds((fbase + f) * 2 * CHUNK, 2 * CHUNK)],
                idx_v.at[b], isem.at[b])

        def gather_desc(j):
            b = lax.rem(j, NBUF)
            ib = lax.rem(lax.div(j, 2), IRING)
            h = lax.rem(j, 2) * CHUNK
            return pltpu.make_async_copy(
                x_hbm.at[idx_v.at[ib, 0, pl.ds(h, CHUNK)]], rows_v.at[b],
                gsem.at[b])

        def scatter_desc(j):
            b = lax.rem(j, NBUF)
            return pltpu.make_async_copy(
                rows_v.at[b], agg_sh.at[dstb_v.at[b]], ssem.at[b])

        for f in range(FPD):
            idx_desc(f).start()
        for f in range(2):
            idx_desc(f).wait()
        for j in range(NBUF - 1):
            gather_desc(j).start()

        @pl.loop(0, CHL)
        def _(j):
            nxt = j + NBUF - 1

            @pl.when(nxt < CHL)
            def _():
                @pl.when(j >= 1)
                def _():
                    scatter_desc(j - 1).wait()

                fn = lax.div(nxt, 2)

                @pl.when((lax.rem(nxt, 2) == 0) & (fn >= 2))
                def _():
                    idx_desc(fn).wait()
                    nf = fn + FPD - 2

                    @pl.when(2 * nf < CHL)
                    def _():
                        idx_desc(nf).start()

                gather_desc(nxt).start()

            gather_desc(j).wait()
            b = lax.rem(j, NBUF)
            ib = lax.rem(lax.div(j, 2), IRING)
            h = lax.rem(j, 2) * CHUNK
            for kk in range(CHUNK // LANES):
                dvec = idx_v[ib, 1, pl.ds(h + kk * LANES, LANES)]
                dstb_v[b, pl.ds(kk * LANES, LANES)] = dvec
                plsc.addupdate_scatter(deg_v, [dvec], ones16)
            scatter_desc(j).start(add=True)

        for i in range(NBUF):
            scatter_desc(CHL - NBUF + i).wait()

        plsc.subcore_barrier()

        pltpu.sync_copy(agg_sh.at[pl.ds(s * STRIDE, SPAN)],
                        agg_out.at[c, pl.ds(s * STRIDE, SPAN)])
        pltpu.sync_copy(deg_v.at[pl.ds(0, N)], deg_out.at[wid, 0])

    return k(x, edge_index, zrows)


def _tc_body(R, G, grid,
             x_r, a0_r, a1_r, deg_r, batch_r, wl_r, wr_r, bl_r, gm_r, bt_r,
             node_r, graph_r, acc_r):
    i = pl.program_id(0)
    deg = jnp.sum(deg_r[...], axis=1, keepdims=True)
    agg = a0_r[...] + a1_r[...]
    mean = agg / jnp.maximum(deg, 1.0)
    conv = (lax.dot_general(mean, wl_r[...], (((1,), (1,)), ((), ())))
            + bl_r[...]
            + lax.dot_general(x_r[...], wr_r[...], (((1,), (1,)), ((), ()))))

    def ln(v):
        mu = jnp.mean(v, axis=-1, keepdims=True)
        var = jnp.mean((v - mu) * (v - mu), axis=-1, keepdims=True)
        return (v - mu) / jnp.sqrt(var + 1e-5) * gm_r[...] + bt_r[...]

    node_r[...] = jnp.maximum(ln(conv + x_r[...]), 0.0)

    oh = (batch_r[...] == lax.broadcasted_iota(jnp.int32, (R, G), 1)
          ).astype(jnp.float32)
    contrib = lax.dot_general(oh, conv, (((0,), (0,)), ((), ())))

    @pl.when(i == 0)
    def _():
        acc_r[...] = contrib

    @pl.when(i > 0)
    def _():
        acc_r[...] = acc_r[...] + contrib

    @pl.when(i == grid - 1)
    def _():
        graph_r[...] = jnp.maximum(ln(acc_r[...]), 0.0)


@functools.partial(jax.jit, static_argnames=("R", "G"))
def _tc_tail(x, a0, a1, deg_p, batch2, W_l, W_r, b_l, gamma, beta, *, R, G):
    N, D = x.shape
    grid = N // R
    row_spec = pl.BlockSpec((R, D), lambda i: (i, 0))
    full_spec = pl.BlockSpec((D, D), lambda i: (0, 0))
    vec_spec = pl.BlockSpec((1, D), lambda i: (0, 0))
    return pl.pallas_call(
        functools.partial(_tc_body, R, G, grid),
        grid=(grid,),
        in_specs=[
            row_spec,
            row_spec,
            row_spec,
            pl.BlockSpec((R, NC * NS), lambda i: (i, 0)),
            pl.BlockSpec((R, 1), lambda i: (i, 0)),
            full_spec,
            full_spec,
            vec_spec,
            vec_spec,
            vec_spec,
        ],
        out_specs=[
            row_spec,
            pl.BlockSpec((G, D), lambda i: (0, 0)),
        ],
        out_shape=[
            jax.ShapeDtypeStruct((N, D), jnp.float32),
            jax.ShapeDtypeStruct((G, D), jnp.float32),
        ],
        scratch_shapes=[pltpu.VMEM((G, D), jnp.float32)],
    )(x, a0, a1, deg_p, batch2, W_l, W_r, b_l, gamma, beta)


def kernel(x, edge_index, batch, W_l, b_l, W_r, gamma, beta):
    N, D = x.shape
    E = edge_index.shape[1]
    G = 16

    assert E == CHUNK * NS * (CH0 + CH1) + CHUNK * 8
    zrows = jnp.zeros((640, D), jnp.float32)

    agg_part, deg_part = _sc_edge_aggregate(
        x, edge_index.astype(jnp.int32), zrows, N=N, D=D)

    node_out, graph_out = _tc_tail(
        x, agg_part[0], agg_part[1], deg_part.reshape(NC * NS, N).T,
        batch.astype(jnp.int32).reshape(N, 1),
        W_l, W_r, b_l.reshape(1, D), gamma.reshape(1, D), beta.reshape(1, D),
        R=1000, G=G)
    return (node_out, graph_out)


# speedup vs baseline: 3.4940x; 1.1346x over previous
import functools

import jax
import jax.numpy as jnp
from jax import lax
from jax.experimental import pallas as pl
from jax.experimental.pallas import tpu as pltpu
from jax.experimental.pallas import tpu_sc as plsc

NC = 2
NS = 16
LANES = 16
CHUNK = 64
NBUF = 4
IRING = 8
FPD = 4
CHB = 156


@functools.partial(jax.jit, static_argnames=("N", "D"))
def _sc_edge_aggregate(x, edge_index, zrows, *, N, D):
    NP = N
    STRIDE = 624
    SPAN = 640
    assert STRIDE * (NS - 1) + SPAN == N

    mesh = plsc.VectorSubcoreMesh(
        core_axis_name="c", subcore_axis_name="s", num_cores=NC,
        num_subcores=NS)

    @functools.partial(
        pl.kernel,
        out_type=[
            jax.ShapeDtypeStruct((NC, N, D), jnp.float32),
            jax.ShapeDtypeStruct((NC * NS, 1, N), jnp.float32),
        ],
        mesh=mesh,
        compiler_params=pltpu.CompilerParams(needs_layout_passes=False),
        scratch_types=[
            pltpu.VMEM((IRING, 2, 2 * CHUNK), jnp.int32),
            pltpu.VMEM((NBUF, CHUNK, D), jnp.float32),
            pltpu.VMEM((NBUF, CHUNK), jnp.int32),
            pltpu.VMEM((NP,), jnp.float32),
            pltpu.VMEM_SHARED((NP, D), jnp.float32),
            pltpu.SemaphoreType.DMA((IRING,)),
            pltpu.SemaphoreType.DMA((NBUF,)),
            pltpu.SemaphoreType.DMA((NBUF,)),
        ],
    )
    def k(x_hbm, edges_hbm, z_hbm, agg_out, deg_out,
          idx_v, rows_v, dstb_v, deg_v, agg_sh, isem, gsem, ssem):
        c = lax.axis_index("c")
        s = lax.axis_index("s")
        wid = c * NS + s
        CHL = CHB + 2 * (s < 2)
        fbase = (c * (NS * CHB + 4) + s * CHB + 2 * jnp.minimum(s, 2)) // 2

        pltpu.sync_copy(z_hbm, agg_sh.at[pl.ds(s * STRIDE, SPAN)])
        zeros16 = jnp.zeros((LANES,), jnp.float32)

        @pl.loop(0, NP // LANES)
        def _(i):
            deg_v[pl.ds(i * LANES, LANES)] = zeros16

        plsc.subcore_barrier()

        ones16 = jnp.ones((LANES,), jnp.float32)

        def idx_desc(f):
            b = lax.rem(f, IRING)
            return pltpu.make_async_copy(
                edges_hbm.at[:, pl.ds((fbase + f) * 2 * CHUNK, 2 * CHUNK)],
                idx_v.at[b], isem.at[b])

        def gather_desc(j):
            b = lax.rem(j, NBUF)
            ib = lax.rem(lax.div(j, 2), IRING)
            h = lax.rem(j, 2) * CHUNK
            return pltpu.make_async_copy(
                x_hbm.at[idx_v.at[ib, 0, pl.ds(h, CHUNK)]], rows_v.at[b],
                gsem.at[b])

        def scatter_desc(j):
            b = lax.rem(j, NBUF)
            return pltpu.make_async_copy(
                rows_v.at[b], agg_sh.at[dstb_v.at[b]], ssem.at[b])

        for f in range(FPD):
            idx_desc(f).start()
        for f in range(2):
            idx_desc(f).wait()
        for j in range(NBUF - 1):
            gather_desc(j).start()

        @pl.loop(0, CHL)
        def _(j):
            nxt = j + NBUF - 1

            @pl.when(nxt < CHL)
            def _():
                @pl.when(j >= 1)
                def _():
                    scatter_desc(j - 1).wait()

                fn = lax.div(nxt, 2)

                @pl.when((lax.rem(nxt, 2) == 0) & (fn >= 2))
                def _():
                    idx_desc(fn).wait()
                    nf = fn + FPD - 2

                    @pl.when(2 * nf < CHL)
                    def _():
                        idx_desc(nf).start()

                gather_desc(nxt).start()

            gather_desc(j).wait()
            b = lax.rem(j, NBUF)
            ib = lax.rem(lax.div(j, 2), IRING)
            h = lax.rem(j, 2) * CHUNK
            for kk in range(CHUNK // LANES):
                dvec = idx_v[ib, 1, pl.ds(h + kk * LANES, LANES)]
                dstb_v[b, pl.ds(kk * LANES, LANES)] = dvec
                plsc.addupdate_scatter(deg_v, [dvec], ones16)
            scatter_desc(j).start(add=True)

        for i in range(NBUF):
            scatter_desc(CHL - NBUF + i).wait()

        plsc.subcore_barrier()

        pltpu.sync_copy(agg_sh.at[pl.ds(s * STRIDE, SPAN)],
                        agg_out.at[c, pl.ds(s * STRIDE, SPAN)])
        pltpu.sync_copy(deg_v.at[pl.ds(0, N)], deg_out.at[wid, 0])

    return k(x, edge_index, zrows)


def _tc_body(R, G, grid,
             x_r, a0_r, a1_r, deg_r, batch_r, wl_r, wr_r, bl_r, gm_r, bt_r,
             node_r, graph_r, acc_r):
    i = pl.program_id(0)
    deg = jnp.sum(deg_r[...], axis=1, keepdims=True)
    agg = a0_r[...] + a1_r[...]
    mean = agg / jnp.maximum(deg, 1.0)
    conv = (lax.dot_general(mean, wl_r[...], (((1,), (1,)), ((), ())))
            + bl_r[...]
            + lax.dot_general(x_r[...], wr_r[...], (((1,), (1,)), ((), ()))))

    def ln(v):
        mu = jnp.mean(v, axis=-1, keepdims=True)
        var = jnp.mean((v - mu) * (v - mu), axis=-1, keepdims=True)
        return (v - mu) / jnp.sqrt(var + 1e-5) * gm_r[...] + bt_r[...]

    node_r[...] = jnp.maximum(ln(conv + x_r[...]), 0.0)

    oh = (batch_r[...] == lax.broadcasted_iota(jnp.int32, (R, G), 1)
          ).astype(jnp.float32)
    contrib = lax.dot_general(oh, conv, (((0,), (0,)), ((), ())))

    @pl.when(i == 0)
    def _():
        acc_r[...] = contrib

    @pl.when(i > 0)
    def _():
        acc_r[...] = acc_r[...] + contrib

    @pl.when(i == grid - 1)
    def _():
        graph_r[...] = jnp.maximum(ln(acc_r[...]), 0.0)


@functools.partial(jax.jit, static_argnames=("R", "G"))
def _tc_tail(x, a0, a1, deg_p, batch2, W_l, W_r, b_l, gamma, beta, *, R, G):
    N, D = x.shape
    grid = N // R
    row_spec = pl.BlockSpec((R, D), lambda i: (i, 0))
    full_spec = pl.BlockSpec((D, D), lambda i: (0, 0))
    vec_spec = pl.BlockSpec((1, D), lambda i: (0, 0))
    return pl.pallas_call(
        functools.partial(_tc_body, R, G, grid),
        grid=(grid,),
        in_specs=[
            row_spec,
            row_spec,
            row_spec,
            pl.BlockSpec((R, NC * NS), lambda i: (i, 0)),
            pl.BlockSpec((R, 1), lambda i: (i, 0)),
            full_spec,
            full_spec,
            vec_spec,
            vec_spec,
            vec_spec,
        ],
        out_specs=[
            row_spec,
            pl.BlockSpec((G, D), lambda i: (0, 0)),
        ],
        out_shape=[
            jax.ShapeDtypeStruct((N, D), jnp.float32),
            jax.ShapeDtypeStruct((G, D), jnp.float32),
        ],
        scratch_shapes=[pltpu.VMEM((G, D), jnp.float32)],
    )(x, a0, a1, deg_p, batch2, W_l, W_r, b_l, gamma, beta)


def kernel(x, edge_index, batch, W_l, b_l, W_r, gamma, beta):
    N, D = x.shape
    E = edge_index.shape[1]
    G = 16

    assert E == CHUNK * (NC * NS * CHB + NC * 4)
    zrows = jnp.zeros((640, D), jnp.float32)

    agg_part, deg_part = _sc_edge_aggregate(
        x, edge_index.astype(jnp.int32), zrows, N=N, D=D)

    node_out, graph_out = _tc_tail(
        x, agg_part[0], agg_part[1], deg_part.reshape(NC * NS, N).T,
        batch.astype(jnp.int32).reshape(N, 1),
        W_l, W_r, b_l.reshape(1, D), gamma.reshape(1, D), beta.reshape(1, D),
        R=1000, G=G)
    return (node_out, graph_out)


# speedup vs baseline: 3.6230x; 1.0369x over previous
import functools

import jax
import jax.numpy as jnp
from jax import lax
from jax.experimental import pallas as pl
from jax.experimental.pallas import tpu as pltpu
from jax.experimental.pallas import tpu_sc as plsc

NC = 2
NS = 16
LANES = 16
CHUNK = 64
NBUF = 4
IRING = 8
FPD = 4
CHB = 156


@functools.partial(jax.jit, static_argnames=("N", "D"))
def _sc_edge_aggregate(x, edge_index, zrows, *, N, D):
    NP = N
    STRIDE = 624
    SPAN = 640
    assert STRIDE * (NS - 1) + SPAN == N

    mesh = plsc.VectorSubcoreMesh(
        core_axis_name="c", subcore_axis_name="s", num_cores=NC,
        num_subcores=NS)

    @functools.partial(
        pl.kernel,
        out_type=[
            jax.ShapeDtypeStruct((NC, N, D), jnp.float32),
            jax.ShapeDtypeStruct((NC * NS, 1, N), jnp.float32),
        ],
        mesh=mesh,
        compiler_params=pltpu.CompilerParams(needs_layout_passes=False),
        scratch_types=[
            pltpu.VMEM((IRING, 2, 2 * CHUNK), jnp.int32),
            pltpu.VMEM((NBUF, CHUNK, D), jnp.float32),
            pltpu.VMEM((NBUF, CHUNK), jnp.int32),
            pltpu.VMEM((NP,), jnp.float32),
            pltpu.VMEM_SHARED((NP, D), jnp.float32),
            pltpu.SemaphoreType.DMA((IRING,)),
            pltpu.SemaphoreType.DMA((NBUF,)),
            pltpu.SemaphoreType.DMA((NBUF,)),
        ],
    )
    def k(x_hbm, edges_hbm, z_hbm, agg_out, deg_out,
          idx_v, rows_v, dstb_v, deg_v, agg_sh, isem, gsem, ssem):
        c = lax.axis_index("c")
        s = lax.axis_index("s")
        wid = c * NS + s
        CHL = CHB + 2 * (s < 2)
        fbase = (c * (NS * CHB + 4) + s * CHB + 2 * jnp.minimum(s, 2)) // 2

        pltpu.sync_copy(z_hbm, agg_sh.at[pl.ds(s * STRIDE, SPAN)])
        zeros16 = jnp.zeros((LANES,), jnp.float32)

        @pl.loop(0, NP // LANES)
        def _(i):
            deg_v[pl.ds(i * LANES, LANES)] = zeros16

        plsc.subcore_barrier()

        ones16 = jnp.ones((LANES,), jnp.float32)

        def idx_desc(f):
            b = lax.rem(f, IRING)
            return pltpu.make_async_copy(
                edges_hbm.at[:, pl.ds((fbase + f) * 2 * CHUNK, 2 * CHUNK)],
                idx_v.at[b], isem.at[b])

        def gather_desc(j):
            b = lax.rem(j, NBUF)
            ib = lax.rem(lax.div(j, 2), IRING)
            h = lax.rem(j, 2) * CHUNK
            return pltpu.make_async_copy(
                x_hbm.at[idx_v.at[ib, 0, pl.ds(h, CHUNK)]], rows_v.at[b],
                gsem.at[b])

        def scatter_desc(j):
            b = lax.rem(j, NBUF)
            return pltpu.make_async_copy(
                rows_v.at[b], agg_sh.at[dstb_v.at[b]], ssem.at[b])

        for f in range(FPD):
            idx_desc(f).start()
        for f in range(2):
            idx_desc(f).wait()
        for j in range(NBUF - 1):
            gather_desc(j).start()

        @pl.loop(0, CHL)
        def _(j):
            nxt = j + NBUF - 1

            @pl.when(nxt < CHL)
            def _():
                @pl.when(j >= 1)
                def _():
                    scatter_desc(j - 1).wait()

                fn = lax.div(nxt, 2)

                @pl.when((lax.rem(nxt, 2) == 0) & (fn >= 2))
                def _():
                    idx_desc(fn).wait()
                    nf = fn + FPD - 2

                    @pl.when(2 * nf < CHL)
                    def _():
                        idx_desc(nf).start()

                gather_desc(nxt).start()

            gather_desc(j).wait()
            b = lax.rem(j, NBUF)
            ib = lax.rem(lax.div(j, 2), IRING)
            h = lax.rem(j, 2) * CHUNK
            for kk in range(CHUNK // LANES):
                dvec = idx_v[ib, 1, pl.ds(h + kk * LANES, LANES)]
                dstb_v[b, pl.ds(kk * LANES, LANES)] = dvec
                plsc.addupdate_scatter(deg_v, [dvec], ones16)
            scatter_desc(j).start(add=True)

        for i in range(NBUF):
            scatter_desc(CHL - NBUF + i).wait()

        plsc.subcore_barrier()

        pltpu.sync_copy(agg_sh.at[pl.ds(s * STRIDE, SPAN)],
                        agg_out.at[c, pl.ds(s * STRIDE, SPAN)])
        pltpu.sync_copy(deg_v.at[pl.ds(0, N)], deg_out.at[wid, 0])

    return k(x, edge_index, zrows)


def _tc_body(R, G, grid,
             x_r, a0_r, a1_r, deg_r, batch_r, wl_r, wr_r, bl_r, gm_r, bt_r,
             node_r, graph_r, acc_r):
    i = pl.program_id(0)
    deg = jnp.sum(deg_r[...], axis=1, keepdims=True)
    agg = a0_r[...] + a1_r[...]
    mean = agg / jnp.maximum(deg, 1.0)
    conv = (lax.dot_general(mean, wl_r[...], (((1,), (1,)), ((), ())))
            + bl_r[...]
            + lax.dot_general(x_r[...], wr_r[...], (((1,), (1,)), ((), ()))))

    def ln(v):
        mu = jnp.mean(v, axis=-1, keepdims=True)
        var = jnp.mean((v - mu) * (v - mu), axis=-1, keepdims=True)
        return (v - mu) / jnp.sqrt(var + 1e-5) * gm_r[...] + bt_r[...]

    node_r[...] = jnp.maximum(ln(conv + x_r[...]), 0.0)

    oh = (batch_r[...] == lax.broadcasted_iota(jnp.int32, (R, G), 1)
          ).astype(jnp.float32)
    contrib = lax.dot_general(oh, conv, (((0,), (0,)), ((), ())))

    @pl.when(i == 0)
    def _():
        acc_r[...] = contrib

    @pl.when(i > 0)
    def _():
        acc_r[...] = acc_r[...] + contrib

    @pl.when(i == grid - 1)
    def _():
        graph_r[...] = jnp.maximum(ln(acc_r[...]), 0.0)


@functools.partial(jax.jit, static_argnames=("R", "G"))
def _tc_tail(x, a0, a1, deg_p, batch2, W_l, W_r, b_l, gamma, beta, *, R, G):
    N, D = x.shape
    grid = N // R
    row_spec = pl.BlockSpec((R, D), lambda i: (i, 0))
    full_spec = pl.BlockSpec((D, D), lambda i: (0, 0))
    vec_spec = pl.BlockSpec((1, D), lambda i: (0, 0))
    return pl.pallas_call(
        functools.partial(_tc_body, R, G, grid),
        grid=(grid,),
        in_specs=[
            row_spec,
            row_spec,
            row_spec,
            pl.BlockSpec((R, NC * NS), lambda i: (i, 0)),
            pl.BlockSpec((R, 1), lambda i: (i, 0)),
            full_spec,
            full_spec,
            vec_spec,
            vec_spec,
            vec_spec,
        ],
        out_specs=[
            row_spec,
            pl.BlockSpec((G, D), lambda i: (0, 0)),
        ],
        out_shape=[
            jax.ShapeDtypeStruct((N, D), jnp.float32),
            jax.ShapeDtypeStruct((G, D), jnp.float32),
        ],
        scratch_shapes=[pltpu.VMEM((G, D), jnp.float32)],
    )(x, a0, a1, deg_p, batch2, W_l, W_r, b_l, gamma, beta)


def kernel(x, edge_index, batch, W_l, b_l, W_r, gamma, beta):
    N, D = x.shape
    E = edge_index.shape[1]
    G = 16

    assert E == CHUNK * (NC * NS * CHB + NC * 4)
    zrows = jnp.zeros((640, D), jnp.float32)

    agg_part, deg_part = _sc_edge_aggregate(
        x, edge_index.astype(jnp.int32), zrows, N=N, D=D)

    node_out, graph_out = _tc_tail(
        x, agg_part[0], agg_part[1], deg_part.reshape(NC * NS, N).T,
        batch.astype(jnp.int32).reshape(N, 1),
        W_l, W_r, b_l.reshape(1, D), gamma.reshape(1, D), beta.reshape(1, D),
        R=2000, G=G)
    return (node_out, graph_out)
